# baseline (speedup 1.0000x reference)
import functools

import jax
import jax.numpy as jnp
from jax import lax
from jax.experimental import pallas as pl
from jax.experimental.pallas import tpu as pltpu
from jax.experimental.pallas import tpu_sc as plsc

_NUM_CORES = 2
_NUM_SUBCORES = 16
_NW = _NUM_CORES * _NUM_SUBCORES
_LANES = 16

_CHUNK_BAGS = 4


def _sc_bag_sum(idx3, table, batch, hist, dim):
    b_per_w = batch // _NW
    n_chunks = b_per_w // _CHUNK_BAGS
    rows_per_chunk = _CHUNK_BAGS * hist

    mesh = plsc.VectorSubcoreMesh(
        core_axis_name="c", subcore_axis_name="s",
        num_cores=_NUM_CORES, num_subcores=_NUM_SUBCORES)

    @functools.partial(
        pl.kernel,
        out_type=jax.ShapeDtypeStruct((batch, dim), jnp.float32),
        mesh=mesh,
        scratch_types=[
            pltpu.VMEM((n_chunks, rows_per_chunk), jnp.int32),
            pltpu.VMEM((rows_per_chunk, dim), jnp.float32),
            pltpu.VMEM((rows_per_chunk, dim), jnp.float32),
            pltpu.VMEM((rows_per_chunk, dim), jnp.float32),
            pltpu.VMEM((rows_per_chunk, dim), jnp.float32),
            pltpu.VMEM((b_per_w, dim), jnp.float32),
            pltpu.SemaphoreType.DMA,
            pltpu.SemaphoreType.DMA,
            pltpu.SemaphoreType.DMA,
            pltpu.SemaphoreType.DMA,
        ],
    )
    def bag_sum(idx_hbm, table_hbm, out_hbm, idx_v, rows0, rows1, rows2,
                rows3, out_v, sem0, sem1, sem2, sem3):
        wid = lax.axis_index("s") * _NUM_CORES + lax.axis_index("c")
        pltpu.sync_copy(idx_hbm.at[wid], idx_v)
        bufs = [(rows0, sem0), (rows1, sem1), (rows2, sem2), (rows3, sem3)]
        nbuf = len(bufs)

        def accum(c, rows_v):
            for bag in range(_CHUNK_BAGS):
                for v in range(dim // _LANES):
                    sl = pl.ds(v * _LANES, _LANES)
                    vals = [rows_v[bag * hist + r, sl] for r in range(hist)]
                    while len(vals) > 1:
                        nxt = [a + b for a, b in zip(vals[::2], vals[1::2])]
                        if len(vals) % 2:
                            nxt.append(vals[-1])
                        vals = nxt
                    out_v[c * _CHUNK_BAGS + bag, sl] = vals[0]

        for q in range(nbuf):
            pltpu.async_copy(table_hbm.at[idx_v.at[q]], bufs[q][0],
                             bufs[q][1])

        def chunk_body(p, carry):
            for q in range(nbuf):
                c = p * nbuf + q
                rows_v, sem = bufs[q]
                pltpu.make_async_copy(table_hbm.at[idx_v.at[c]], rows_v,
                                      sem).wait()
                accum(c, rows_v)

                @pl.when(c + nbuf < n_chunks)
                def _():
                    pltpu.async_copy(table_hbm.at[idx_v.at[c + nbuf]],
                                     rows_v, sem)

            return carry

        lax.fori_loop(0, n_chunks // nbuf, chunk_body, 0, unroll=False)
        pltpu.sync_copy(out_v, out_hbm.at[pl.ds(wid * b_per_w, b_per_w)])

    return bag_sum(idx3, table)


def _tc_head(summed, idxT, wt_bf16, batch, hist, dim, classes, tb):
    def body(summed_ref, idxT_ref, wt_ref, out_ref):
        cnt = jnp.sum((idxT_ref[...] != 0).astype(jnp.float32), axis=0,
                      keepdims=True)
        recip = 1.0 / jnp.maximum(cnt, 1.0)
        pooledT = jnp.transpose(summed_ref[...]) * recip
        logits = lax.dot_general(
            wt_ref[...], pooledT.astype(jnp.bfloat16),
            (((1,), (0,)), ((), ())),
            preferred_element_type=jnp.float32)
        m = jnp.max(logits, axis=0, keepdims=True)
        shifted = logits - m
        lse = jnp.log(jnp.sum(jnp.exp(shifted), axis=0, keepdims=True))
        out_ref[...] = shifted - lse

    grid = (batch // tb,)
    return pl.pallas_call(
        body,
        grid=grid,
        in_specs=[
            pl.BlockSpec((tb, dim), lambda i: (i, 0)),
            pl.BlockSpec((hist, tb), lambda i: (0, i)),
            pl.BlockSpec((classes, dim), lambda i: (0, 0)),
        ],
        out_specs=pl.BlockSpec((classes, tb), lambda i: (0, i)),
        out_shape=jax.ShapeDtypeStruct((classes, batch), jnp.float32),
        compiler_params=pltpu.CompilerParams(
            dimension_semantics=("arbitrary",),
            vmem_limit_bytes=100 * 1024 * 1024,
        ),
    )(summed, idxT, wt_bf16)


def kernel(indexes, embedding_weight, head_weight):
    batch, hist = indexes.shape
    vocab, dim = embedding_weight.shape
    classes = head_weight.shape[0]

    idx = indexes.astype(jnp.int32)
    b_per_w = batch // _NW
    n_chunks = b_per_w // _CHUNK_BAGS
    idx3 = idx.reshape(_NW, n_chunks, _CHUNK_BAGS * hist)

    summed = _sc_bag_sum(idx3, embedding_weight, batch, hist, dim)

    wt = head_weight.astype(jnp.bfloat16)
    out_t = _tc_head(summed, idx.T, wt, batch, hist, dim, classes, tb=256)
    return out_t.T


# speedup vs baseline: 1.0723x; 1.0723x over previous
import functools

import jax
import jax.numpy as jnp
from jax import lax
from jax.experimental import pallas as pl
from jax.experimental.pallas import tpu as pltpu
from jax.experimental.pallas import tpu_sc as plsc

_NUM_CORES = 2
_NUM_SUBCORES = 16
_NW = _NUM_CORES * _NUM_SUBCORES
_LANES = 16

_CHUNK_BAGS = 4


def _sc_bag_sum(idx3, table, batch, hist, dim):
    b_per_w = batch // _NW
    n_chunks = b_per_w // _CHUNK_BAGS
    rows_per_chunk = _CHUNK_BAGS * hist

    mesh = plsc.VectorSubcoreMesh(
        core_axis_name="c", subcore_axis_name="s",
        num_cores=_NUM_CORES, num_subcores=_NUM_SUBCORES)

    @functools.partial(
        pl.kernel,
        out_type=jax.ShapeDtypeStruct((batch, dim), jnp.float32),
        mesh=mesh,
        scratch_types=[
            pltpu.VMEM((n_chunks, rows_per_chunk), jnp.int32),
            pltpu.VMEM((rows_per_chunk, dim), jnp.float32),
            pltpu.VMEM((rows_per_chunk, dim), jnp.float32),
            pltpu.VMEM((b_per_w, dim), jnp.float32),
            pltpu.SemaphoreType.DMA,
            pltpu.SemaphoreType.DMA,
        ],
    )
    def bag_sum(idx_hbm, table_hbm, out_hbm, idx_v, rows0, rows1, out_v,
                sem0, sem1):
        wid = lax.axis_index("s") * _NUM_CORES + lax.axis_index("c")
        pltpu.sync_copy(idx_hbm.at[wid], idx_v)
        bufs = [(rows0, sem0), (rows1, sem1)]
        nbuf = len(bufs)

        def accum(c, rows_v):
            for bag in range(_CHUNK_BAGS):
                for v in range(dim // _LANES):
                    sl = pl.ds(v * _LANES, _LANES)
                    vals = [rows_v[bag * hist + r, sl] for r in range(hist)]
                    while len(vals) > 1:
                        nxt = [a + b for a, b in zip(vals[::2], vals[1::2])]
                        if len(vals) % 2:
                            nxt.append(vals[-1])
                        vals = nxt
                    out_v[c * _CHUNK_BAGS + bag, sl] = vals[0]

        for q in range(nbuf):
            pltpu.async_copy(table_hbm.at[idx_v.at[q]], bufs[q][0],
                             bufs[q][1])

        def chunk_body(p, carry):
            for q in range(nbuf):
                c = p * nbuf + q
                rows_v, sem = bufs[q]
                pltpu.make_async_copy(table_hbm.at[idx_v.at[c]], rows_v,
                                      sem).wait()
                accum(c, rows_v)

                @pl.when(c + nbuf < n_chunks)
                def _():
                    pltpu.async_copy(table_hbm.at[idx_v.at[c + nbuf]],
                                     rows_v, sem)

            return carry

        lax.fori_loop(0, n_chunks // nbuf, chunk_body, 0, unroll=False)
        pltpu.sync_copy(out_v, out_hbm.at[pl.ds(wid * b_per_w, b_per_w)])

    return bag_sum(idx3, table)


def _tc_head(summed, idxT, wt_bf16, batch, hist, dim, classes, tb):
    def body(summed_ref, idxT_ref, wt_ref, out_ref):
        cnt = jnp.sum((idxT_ref[...] != 0).astype(jnp.float32), axis=0,
                      keepdims=True)
        recip = 1.0 / jnp.maximum(cnt, 1.0)
        pooledT = jnp.transpose(summed_ref[...]) * recip
        logits = lax.dot_general(
            wt_ref[...], pooledT.astype(jnp.bfloat16),
            (((1,), (0,)), ((), ())),
            preferred_element_type=jnp.float32)
        m = jnp.max(logits, axis=0, keepdims=True)
        lse = jnp.log(jnp.sum(jnp.exp(logits - m), axis=0, keepdims=True))
        out_ref[...] = logits - (m + lse)

    grid = (batch // tb,)
    return pl.pallas_call(
        body,
        grid=grid,
        in_specs=[
            pl.BlockSpec((tb, dim), lambda i: (i, 0)),
            pl.BlockSpec((hist, tb), lambda i: (0, i)),
            pl.BlockSpec((classes, dim), lambda i: (0, 0)),
        ],
        out_specs=pl.BlockSpec((classes, tb), lambda i: (0, i)),
        out_shape=jax.ShapeDtypeStruct((classes, batch), jnp.float32),
        compiler_params=pltpu.CompilerParams(
            dimension_semantics=("parallel",),
            vmem_limit_bytes=100 * 1024 * 1024,
        ),
    )(summed, idxT, wt_bf16)


def kernel(indexes, embedding_weight, head_weight):
    batch, hist = indexes.shape
    vocab, dim = embedding_weight.shape
    classes = head_weight.shape[0]

    idx = indexes.astype(jnp.int32)
    b_per_w = batch // _NW
    n_chunks = b_per_w // _CHUNK_BAGS
    idx3 = idx.reshape(_NW, n_chunks, _CHUNK_BAGS * hist)

    summed = _sc_bag_sum(idx3, embedding_weight, batch, hist, dim)

    wt = head_weight.astype(jnp.bfloat16)
    out_t = _tc_head(summed, idx.T, wt, batch, hist, dim, classes, tb=256)
    return out_t.T


# speedup vs baseline: 1.1981x; 1.1173x over previous
import functools

import jax
import jax.numpy as jnp
from jax import lax
from jax.experimental import pallas as pl
from jax.experimental.pallas import tpu as pltpu
from jax.experimental.pallas import tpu_sc as plsc

_NUM_CORES = 2
_NUM_SUBCORES = 16
_NW = _NUM_CORES * _NUM_SUBCORES
_LANES = 16

_CHUNK_BAGS = 4

_NSPLIT = 2


def _sc_bag_sum(idx3, table, batch, hist, dim):
    b_per_w = batch // _NW
    n_chunks = b_per_w // _CHUNK_BAGS
    rows_per_chunk = _CHUNK_BAGS * hist

    mesh = plsc.VectorSubcoreMesh(
        core_axis_name="c", subcore_axis_name="s",
        num_cores=_NUM_CORES, num_subcores=_NUM_SUBCORES)

    @functools.partial(
        pl.kernel,
        out_type=jax.ShapeDtypeStruct((batch, dim), jnp.float32),
        mesh=mesh,
        scratch_types=[
            pltpu.VMEM((n_chunks, rows_per_chunk), jnp.int32),
            pltpu.VMEM((rows_per_chunk, dim), jnp.float32),
            pltpu.VMEM((rows_per_chunk, dim), jnp.float32),
            pltpu.VMEM((b_per_w, dim), jnp.float32),
            pltpu.SemaphoreType.DMA,
            pltpu.SemaphoreType.DMA,
        ],
    )
    def bag_sum(idx_hbm, table_hbm, out_hbm, idx_v, rows0, rows1, out_v,
                sem0, sem1):
        wid = lax.axis_index("s") * _NUM_CORES + lax.axis_index("c")
        pltpu.sync_copy(idx_hbm.at[wid], idx_v)
        bufs = [(rows0, sem0), (rows1, sem1)]
        nbuf = len(bufs)

        def accum(c, rows_v):
            for bag in range(_CHUNK_BAGS):
                for v in range(dim // _LANES):
                    sl = pl.ds(v * _LANES, _LANES)
                    vals = [rows_v[bag * hist + r, sl] for r in range(hist)]
                    while len(vals) > 1:
                        nxt = [a + b for a, b in zip(vals[::2], vals[1::2])]
                        if len(vals) % 2:
                            nxt.append(vals[-1])
                        vals = nxt
                    out_v[c * _CHUNK_BAGS + bag, sl] = vals[0]

        for q in range(nbuf):
            pltpu.async_copy(table_hbm.at[idx_v.at[q]], bufs[q][0],
                             bufs[q][1])

        def chunk_body(p, carry):
            for q in range(nbuf):
                c = p * nbuf + q
                rows_v, sem = bufs[q]
                pltpu.make_async_copy(table_hbm.at[idx_v.at[c]], rows_v,
                                      sem).wait()
                accum(c, rows_v)

                @pl.when(c + nbuf < n_chunks)
                def _():
                    pltpu.async_copy(table_hbm.at[idx_v.at[c + nbuf]],
                                     rows_v, sem)

            return carry

        lax.fori_loop(0, n_chunks // nbuf, chunk_body, 0, unroll=False)
        pltpu.sync_copy(out_v, out_hbm.at[pl.ds(wid * b_per_w, b_per_w)])

    return bag_sum(idx3, table)


def _tc_head_body(summed_ref, idxT_ref, wt_ref, out_ref):
    cnt = jnp.sum((idxT_ref[...] != 0).astype(jnp.float32), axis=0,
                  keepdims=True)
    recip = 1.0 / jnp.maximum(cnt, 1.0)
    pooledT = jnp.transpose(summed_ref[...]) * recip
    logits = lax.dot_general(
        wt_ref[...], pooledT.astype(jnp.bfloat16),
        (((1,), (0,)), ((), ())),
        preferred_element_type=jnp.float32)
    m = jnp.max(logits, axis=0, keepdims=True)
    lse = jnp.log(jnp.sum(jnp.exp(logits - m), axis=0, keepdims=True))
    out_ref[...] = logits - (m + lse)


def _tc_head_first(summed, idxT, wt_bf16, batch, hist, dim, classes, tb):
    nt = summed.shape[0] // tb

    def body(summed_ref, idxT_ref, wt_ref, out_ref):
        _tc_head_body(summed_ref, idxT_ref, wt_ref, out_ref)

    return pl.pallas_call(
        body,
        grid=(nt,),
        in_specs=[
            pl.BlockSpec((tb, dim), lambda i: (i, 0)),
            pl.BlockSpec((hist, tb), lambda i: (0, i)),
            pl.BlockSpec((classes, dim), lambda i: (0, 0)),
        ],
        out_specs=pl.BlockSpec((classes, tb), lambda i: (0, i)),
        out_shape=jax.ShapeDtypeStruct((classes, batch), jnp.float32),
        compiler_params=pltpu.CompilerParams(
            dimension_semantics=("parallel",),
            vmem_limit_bytes=100 * 1024 * 1024,
        ),
    )(summed, idxT, wt_bf16)


def _tc_head_next(prev_out, summed, idxT, wt_bf16, tile0, hist, dim,
                  classes, tb):
    nt = summed.shape[0] // tb
    batch = prev_out.shape[1]

    def body(prev_ref, summed_ref, idxT_ref, wt_ref, out_ref):
        del prev_ref
        _tc_head_body(summed_ref, idxT_ref, wt_ref, out_ref)

    return pl.pallas_call(
        body,
        grid=(nt,),
        in_specs=[
            pl.BlockSpec(memory_space=pl.ANY),
            pl.BlockSpec((tb, dim), lambda i: (i, 0)),
            pl.BlockSpec((hist, tb), lambda i: (0, i)),
            pl.BlockSpec((classes, dim), lambda i: (0, 0)),
        ],
        out_specs=pl.BlockSpec((classes, tb),
                               lambda i, tile0=tile0: (0, i + tile0)),
        out_shape=jax.ShapeDtypeStruct((classes, batch), jnp.float32),
        input_output_aliases={0: 0},
        compiler_params=pltpu.CompilerParams(
            dimension_semantics=("parallel",),
            vmem_limit_bytes=100 * 1024 * 1024,
        ),
    )(prev_out, summed, idxT, wt_bf16)


def kernel(indexes, embedding_weight, head_weight):
    batch, hist = indexes.shape
    vocab, dim = embedding_weight.shape
    classes = head_weight.shape[0]
    tb = 256

    idx = indexes.astype(jnp.int32)
    idxT = idx.T
    wt = head_weight.astype(jnp.bfloat16)

    hb = batch // _NSPLIT
    b_per_w = hb // _NW
    n_chunks = b_per_w // _CHUNK_BAGS

    summed = []
    for h in range(_NSPLIT):
        idx3 = idx[h * hb:(h + 1) * hb].reshape(
            _NW, n_chunks, _CHUNK_BAGS * hist)
        summed.append(_sc_bag_sum(idx3, embedding_weight, hb, hist, dim))

    out_t = _tc_head_first(summed[0], idxT[:, :hb], wt, batch, hist, dim,
                           classes, tb)
    for h in range(1, _NSPLIT):
        out_t = _tc_head_next(out_t, summed[h],
                              idxT[:, h * hb:(h + 1) * hb], wt,
                              h * (hb // tb), hist, dim, classes, tb)
    return out_t.T
